# baseline (speedup 1.0000x reference)
import dataclasses
import functools
import jax
import jax.numpy as jnp
import numpy as np
from jax import lax
from jax.experimental import pallas as pl
from jax.experimental.pallas import tpu as pltpu
from jax.experimental.pallas import tpu_sc as plsc

N = 10000
E = 320000
D = 128
R = 8
D_OUT = 64

NUM_CORES = 2
NUM_SUBCORES = 16
NUM_TILES = NUM_CORES * NUM_SUBCORES
CHUNK = 128
NCHUNKS = E // CHUNK
T0 = 80
T1 = 80

STRIPE = 632
LAST_STRIPE = N - 15 * STRIPE

def _proj_and_shift(xb, wqk_ref, xqk_ref, cv_ref):
    xqk = jnp.dot(xb, wqk_ref[...].astype(jnp.bfloat16),
                  preferred_element_type=jnp.float32)
    xqk_ref[...] = xqk
    m = jnp.max(xqk[:, 0:R]) + jnp.max(xqk[:, R:2 * R])
    cv = jnp.where(m >= 0, m, 0.2 * m)
    cv_ref[0, :] = jnp.full((16,), cv, jnp.float32)


def _t1_body(x_ref, w_ref, wqk_ref, xwb_ref, xqk_ref, cv_ref, xb_ref):
    r = pl.program_id(0)

    @pl.when(r == 0)
    def _():
        xb_ref[...] = x_ref[...].astype(jnp.bfloat16)
        _proj_and_shift(xb_ref[...], wqk_ref, xqk_ref, cv_ref)

    wb = w_ref[0].astype(jnp.bfloat16)
    xwb_ref[...] = jnp.dot(xb_ref[...], wb,
                           preferred_element_type=jnp.float32)


def _transform1(x, w, wqk):
    return pl.pallas_call(
        _t1_body,
        grid=(R,),
        in_specs=[
            pl.BlockSpec((N, D), lambda r: (0, 0)),
            pl.BlockSpec((1, D, D), lambda r: (r, 0, 0)),
            pl.BlockSpec((D, 2 * R), lambda r: (0, 0)),
        ],
        out_specs=[
            pl.BlockSpec((N, D), lambda r: (r, 0)),
            pl.BlockSpec((N, 2 * R), lambda r: (0, 0)),
            pl.BlockSpec((1, 16), lambda r: (0, 0)),
        ],
        out_shape=[
            jax.ShapeDtypeStruct((R * N, D), jnp.float32),
            jax.ShapeDtypeStruct((N, 2 * R), jnp.float32),
            jax.ShapeDtypeStruct((1, 16), jnp.float32),
        ],
        scratch_shapes=[pltpu.VMEM((N, D), jnp.bfloat16)],
    )(x, w, wqk)


def _t2_body(num_ref, den_ref, b_ref, w_ref, wqk_ref,
             xwb_ref, xqk_ref, cv_ref, hb_ref):
    r = pl.program_id(0)

    @pl.when(r == 0)
    def _():
        ns = num_ref[0] + num_ref[1]
        d = jnp.sum(den_ref[...], axis=0)
        h = ns / (d[:, None] + 1e-16) + b_ref[...]
        hb_ref[...] = jnp.maximum(h, 0.0).astype(jnp.bfloat16)
        _proj_and_shift(hb_ref[...], wqk_ref, xqk_ref, cv_ref)

    wb = w_ref[0].astype(jnp.bfloat16)
    xwb_ref[...] = jnp.dot(hb_ref[...], wb,
                           preferred_element_type=jnp.float32)


def _transform2(num, den, b, w, wqk):
    return pl.pallas_call(
        _t2_body,
        grid=(R,),
        in_specs=[
            pl.BlockSpec((2, N, D), lambda r: (0, 0, 0)),
            pl.BlockSpec((NUM_TILES, N), lambda r: (0, 0)),
            pl.BlockSpec((1, D), lambda r: (0, 0)),
            pl.BlockSpec((1, D, D), lambda r: (r, 0, 0)),
            pl.BlockSpec((D, 2 * R), lambda r: (0, 0)),
        ],
        out_specs=[
            pl.BlockSpec((N, D), lambda r: (r, 0)),
            pl.BlockSpec((N, 2 * R), lambda r: (0, 0)),
            pl.BlockSpec((1, 16), lambda r: (0, 0)),
        ],
        out_shape=[
            jax.ShapeDtypeStruct((R * N, D), jnp.float32),
            jax.ShapeDtypeStruct((N, 2 * R), jnp.float32),
            jax.ShapeDtypeStruct((1, 16), jnp.float32),
        ],
        scratch_shapes=[pltpu.VMEM((N, D), jnp.bfloat16)],
    )(num, den, b.reshape(1, D), w, wqk)


def _t3_body(num_ref, den_ref, b_ref, lw_ref, lb_ref, out_ref):
    ns = num_ref[0] + num_ref[1]
    d = jnp.sum(den_ref[...], axis=0)
    h = jnp.maximum(ns / (d[:, None] + 1e-16) + b_ref[...], 0.0)
    pooled = jnp.sum(h, axis=0, keepdims=True) * (1.0 / N)
    logits = jnp.dot(pooled, lw_ref[...],
                     preferred_element_type=jnp.float32) + lb_ref[...]
    m = jnp.max(logits)
    z = logits - m
    out_ref[...] = z - jnp.log(jnp.sum(jnp.exp(z)))


def _final(num, den, b, lin_w, lin_b):
    return pl.pallas_call(
        _t3_body,
        grid=(1,),
        in_specs=[
            pl.BlockSpec((2, N, D), lambda i: (0, 0, 0)),
            pl.BlockSpec((NUM_TILES, N), lambda i: (0, 0)),
            pl.BlockSpec((1, D), lambda i: (0, 0)),
            pl.BlockSpec((D, D_OUT), lambda i: (0, 0)),
            pl.BlockSpec((1, D_OUT), lambda i: (0, 0)),
        ],
        out_specs=pl.BlockSpec((1, D_OUT), lambda i: (0, 0)),
        out_shape=jax.ShapeDtypeStruct((1, D_OUT), jnp.float32),
    )(num, den, b.reshape(1, D), lin_w, lin_b.reshape(1, D_OUT))


def _edge_body(ei_hbm, typ_hbm, xwb_hbm, xqk_hbm, cvec_hbm,
               z128_hbm, zn_hbm,
               num_out, den_out,
               sv0, sv1, tv0, tv1, qv0, qv1, kv0, kv1, rv0, rv1,
               dv0, dv1, sd0, sd1,
               sq0, sq1, sk0, sk1, eav, rb0, rb1, denv, cvv,
               num_sh,
               si0, si1, sg0, sg1, ss0, ss1):
    core = lax.axis_index("c")
    sid = lax.axis_index("s")
    wid = sid * NUM_CORES + core

    start = jnp.where(core == 0, sid * T0, 16 * T0 + sid * T1)
    tcap = jnp.where(core == 0, T0, T1)
    count = jnp.maximum(0, jnp.minimum(tcap, NCHUNKS - start))
    half = count // 2

    sv = (sv0, sv1)
    tv = (tv0, tv1)
    qv = (qv0, qv1)
    kv = (kv0, kv1)
    rv = (rv0, rv1)
    dv = (dv0, dv1)
    sd = (sd0, sd1)
    sq = (sq0, sq1)
    sk = (sk0, sk1)
    rb = (rb0, rb1)
    si = (si0, si1)
    sg = (sg0, sg1)
    ss = (ss0, ss1)

    def idx_copies(i, p):
        b = (start + i) * CHUNK
        return (pltpu.make_async_copy(ei_hbm.at[0, pl.ds(b, CHUNK)], sv[p], si[p]),
                pltpu.make_async_copy(ei_hbm.at[1, pl.ds(b, CHUNK)], dv[p], si[p]),
                pltpu.make_async_copy(typ_hbm.at[pl.ds(b, CHUNK)], tv[p], si[p]))

    def flat_idx(p):
        for jj in range(0, CHUNK, 16):
            sl = pl.ds(jj, 16)
            t = tv[p][sl]
            qv[p][sl] = dv[p][sl] * (2 * R) + t
            kv[p][sl] = sv[p][sl] * (2 * R) + (t + R)
            rv[p][sl] = t * N + sv[p][sl]

    def gather_copies(p):
        return (pltpu.make_async_copy(xqk_hbm.at[qv[p]], sq[p], sg[p]),
                pltpu.make_async_copy(xqk_hbm.at[kv[p]], sk[p], sg[p]),
                pltpu.make_async_copy(xwb_hbm.at[rv[p]], rb[p], sg[p]))

    stripe0 = sid * STRIPE

    @pl.when(sid < 15)
    def _():
        pltpu.sync_copy(z128_hbm.at[pl.ds(stripe0, STRIPE)],
                        num_sh.at[pl.ds(stripe0, STRIPE)])

    @pl.when(sid == 15)
    def _():
        pltpu.sync_copy(z128_hbm.at[pl.ds(15 * STRIPE, LAST_STRIPE)],
                        num_sh.at[pl.ds(15 * STRIPE, LAST_STRIPE)])

    pltpu.sync_copy(zn_hbm, denv)
    pltpu.sync_copy(cvec_hbm.at[0], cvv)
    plsc.subcore_barrier()

    def process(i, j, p):
        for c in gather_copies(p):
            c.wait()
        cv = cvv[...]
        for jj in range(0, CHUNK, 16):
            sl = pl.ds(jj, 16)
            z = sq[p][sl] + sk[p][sl]
            alpha = jnp.maximum(z, 0.2 * z)
            ea = jnp.exp(alpha - cv)
            eav[sl] = ea
            d16 = dv[p][sl]
            plsc.addupdate_scatter(denv, [d16], ea)
            sd[p][sl] = d16

        @pl.when(j < half - 1)
        def _():
            for c in idx_copies(i + 2, p):
                c.start()

        @pl.loop(0, CHUNK)
        def _(e):
            splat = plsc.load_gather(eav, [jnp.full((16,), e, jnp.int32)])
            for kk in range(0, D, 16):
                ksl = pl.ds(kk, 16)
                rb[p][e, ksl] = rb[p][e, ksl] * splat

        pltpu.async_copy(rb[p], num_sh.at[sd[p]], ss[p], add=True)

    @pl.when(half > 0)
    def _():
        for c in idx_copies(0, 0):
            c.start()
        for c in idx_copies(0, 0):
            c.wait()
        flat_idx(0)
        for c in gather_copies(0):
            c.start()
        for c in idx_copies(1, 1):
            c.start()

        @pl.loop(0, half)
        def _(j):
            i = 2 * j
            for c in idx_copies(i + 1, 1):
                c.wait()
            flat_idx(1)

            @pl.when(j > 0)
            def _():
                pltpu.make_async_copy(rb[1], num_sh.at[sd[1]], ss[1]).wait()

            for c in gather_copies(1):
                c.start()
            process(i, j, 0)

            @pl.when(j < half - 1)
            def _():
                for c in idx_copies(i + 2, 0):
                    c.wait()
                flat_idx(0)

            pltpu.make_async_copy(rb[0], num_sh.at[sd[0]], ss[0]).wait()

            @pl.when(j < half - 1)
            def _():
                for c in gather_copies(0):
                    c.start()

            process(i + 1, j, 1)

        pltpu.make_async_copy(rb[1], num_sh.at[sd[1]], ss[1]).wait()

    plsc.subcore_barrier()

    @pl.when(sid < 15)
    def _():
        pltpu.sync_copy(num_sh.at[pl.ds(stripe0, STRIPE)],
                        num_out.at[core, pl.ds(stripe0, STRIPE)])

    @pl.when(sid == 15)
    def _():
        pltpu.sync_copy(num_sh.at[pl.ds(15 * STRIPE, LAST_STRIPE)],
                        num_out.at[core, pl.ds(15 * STRIPE, LAST_STRIPE)])

    pltpu.sync_copy(denv, den_out.at[wid])


@jax.jit
def _edge_pass(ei, typ, xwb, xqk_flat, cvec, z128, zn):
    mesh = plsc.VectorSubcoreMesh(core_axis_name="c", subcore_axis_name="s")
    cp = pltpu.CompilerParams()
    if "needs_layout_passes" in pltpu.CompilerParams.__dataclass_fields__:
        cp = dataclasses.replace(cp, needs_layout_passes=False)
    f = pl.kernel(
        _edge_body,
        out_type=[
            jax.ShapeDtypeStruct((NUM_CORES, N, D), jnp.float32),
            jax.ShapeDtypeStruct((NUM_TILES, N), jnp.float32),
        ],
        mesh=mesh,
        scratch_types=[
            pltpu.VMEM((CHUNK,), jnp.int32),
            pltpu.VMEM((CHUNK,), jnp.int32),
            pltpu.VMEM((CHUNK,), jnp.int32),
            pltpu.VMEM((CHUNK,), jnp.int32),
            pltpu.VMEM((CHUNK,), jnp.int32),
            pltpu.VMEM((CHUNK,), jnp.int32),
            pltpu.VMEM((CHUNK,), jnp.int32),
            pltpu.VMEM((CHUNK,), jnp.int32),
            pltpu.VMEM((CHUNK,), jnp.int32),
            pltpu.VMEM((CHUNK,), jnp.int32),
            pltpu.VMEM((CHUNK,), jnp.int32),
            pltpu.VMEM((CHUNK,), jnp.int32),
            pltpu.VMEM((CHUNK,), jnp.int32),
            pltpu.VMEM((CHUNK,), jnp.int32),
            pltpu.VMEM((CHUNK,), jnp.float32),
            pltpu.VMEM((CHUNK,), jnp.float32),
            pltpu.VMEM((CHUNK,), jnp.float32),
            pltpu.VMEM((CHUNK,), jnp.float32),
            pltpu.VMEM((CHUNK,), jnp.float32),
            pltpu.VMEM((CHUNK, D), jnp.float32),
            pltpu.VMEM((CHUNK, D), jnp.float32),
            pltpu.VMEM((N,), jnp.float32),
            pltpu.VMEM((16,), jnp.float32),
            pltpu.VMEM_SHARED((N, D), jnp.float32),
            pltpu.SemaphoreType.DMA,
            pltpu.SemaphoreType.DMA,
            pltpu.SemaphoreType.DMA,
            pltpu.SemaphoreType.DMA,
            pltpu.SemaphoreType.DMA,
            pltpu.SemaphoreType.DMA,
        ],
        compiler_params=cp,
    )
    return f(ei, typ, xwb, xqk_flat, cvec, z128, zn)


def kernel(x, edge_index, edge_type, w1, q1, k1, b1, w2, q2, k2, b2,
           lin_w, lin_b):
    ei = edge_index.astype(jnp.int32)
    typ = edge_type.astype(jnp.int32)
    wqk1 = jnp.concatenate([jnp.matmul(w1, q1)[:, :, 0].T,
                            jnp.matmul(w1, k1)[:, :, 0].T], axis=1)
    wqk2 = jnp.concatenate([jnp.matmul(w2, q2)[:, :, 0].T,
                            jnp.matmul(w2, k2)[:, :, 0].T], axis=1)
    z128 = jnp.zeros((N, D), jnp.float32)
    zn = jnp.zeros((N,), jnp.float32)

    xwb1, xqk1, c1 = _transform1(x, w1, wqk1)
    num1, den1 = _edge_pass(ei, typ, xwb1, xqk1.reshape(-1), c1, z128, zn)

    xwb2, xqk2, c2 = _transform2(num1, den1, b1, w2, wqk2)
    num2, den2 = _edge_pass(ei, typ, xwb2, xqk2.reshape(-1), c2, z128, zn)

    return _final(num2, den2, b2, lin_w, lin_b)
